# Initial kernel scaffold; baseline (speedup 1.0000x reference)
import jax
import jax.numpy as jnp
from jax import lax
from jax.experimental import pallas as pl
from jax.experimental.pallas import tpu as pltpu
from jax.experimental.pallas import tpu_sc as plsc

M = 100000
D = 128
B = 16384

NC = 2
NS = 16
L = 16

ROWS_PER_CORE = M // NC
WIN = 6400
NPASS = -(-ROWS_PER_CORE // WIN)
IDX_PER_TILE = B // NS
NVREG = IDX_PER_TILE // L
KCH = 64
SEL_CAP = IDX_PER_TILE + KCH
NDUMMY = NS


def _body(mem_hbm, idx_hbm, val_hbm, out_hbm,
          idx_v, sel_pos, sel_off, stage_pos, stage_off, upd, shared, sem):
    c = lax.axis_index("c")
    s = lax.axis_index("s")

    pltpu.sync_copy(idx_hbm.at[pl.ds(s * IDX_PER_TILE, IDX_PER_TILE)], idx_v)

    for p in range(NPASS):
        rows_c = min(WIN, ROWS_PER_CORE - p * WIN)
        rows_pt = rows_c // NS
        lo = c * ROWS_PER_CORE + p * WIN
        row0 = lo + s * rows_pt

        pltpu.sync_copy(mem_hbm.at[pl.ds(row0, rows_pt)],
                        shared.at[pl.ds(s * rows_pt, rows_pt)])
        plsc.subcore_barrier()

        def scan_body(i, cnt, lo=lo, rows_c=rows_c):
            v = idx_v[pl.ds(i * L, L)]
            m = (v >= lo) & (v < lo + rows_c)
            pos = lax.iota(jnp.int32, L) + (s * IDX_PER_TILE + i * L)
            plsc.store_compressed(sel_pos.at[pl.ds(cnt, L)], pos, mask=m)
            plsc.store_compressed(sel_off.at[pl.ds(cnt, L)], v - lo, mask=m)
            return cnt + jnp.sum(m.astype(jnp.int32))

        cnt = lax.fori_loop(0, NVREG, scan_body, jnp.int32(0))

        dummy_pos = jnp.full((L,), s * IDX_PER_TILE, jnp.int32)
        dummy_off = jnp.full((L,), WIN + s, jnp.int32)
        for j in range(KCH // L):
            sel_pos[pl.ds(cnt + j * L, L)] = dummy_pos
            sel_off[pl.ds(cnt + j * L, L)] = dummy_off

        def dma_body(i, carry):
            o = i * KCH
            pltpu.sync_copy(sel_pos.at[pl.ds(o, KCH)], stage_pos)
            pltpu.sync_copy(sel_off.at[pl.ds(o, KCH)], stage_off)
            pltpu.async_copy(val_hbm.at[stage_pos], upd, sem).wait()
            pltpu.sync_copy(upd, shared.at[stage_off], add=True)
            return carry

        n_it = (cnt + (KCH - 1)) // KCH
        lax.fori_loop(0, n_it, dma_body, jnp.int32(0))
        plsc.subcore_barrier()

        pltpu.sync_copy(shared.at[pl.ds(s * rows_pt, rows_pt)],
                        out_hbm.at[pl.ds(row0, rows_pt)])


_sc_scatter_add = pl.kernel(
    _body,
    out_type=jax.ShapeDtypeStruct((M, D), jnp.float32),
    mesh=plsc.VectorSubcoreMesh(core_axis_name="c", subcore_axis_name="s"),
    scratch_types=[
        pltpu.VMEM((IDX_PER_TILE,), jnp.int32),
        pltpu.VMEM((SEL_CAP,), jnp.int32),
        pltpu.VMEM((SEL_CAP,), jnp.int32),
        pltpu.VMEM((KCH,), jnp.int32),
        pltpu.VMEM((KCH,), jnp.int32),
        pltpu.VMEM((KCH, D), jnp.float32),
        pltpu.VMEM_SHARED((WIN + NDUMMY, D), jnp.float32),
        pltpu.SemaphoreType.DMA,
    ],
)


def kernel(mem, idx, val):
    return _sc_scatter_add(mem, idx.astype(jnp.int32), val)


# speedup vs baseline: 1.0801x; 1.0801x over previous
import jax
import jax.numpy as jnp
from jax import lax
from jax.experimental import pallas as pl
from jax.experimental.pallas import tpu as pltpu
from jax.experimental.pallas import tpu_sc as plsc

M = 100000
D = 128
B = 16384

NC = 2
NS = 16
L = 16

ROWS_PER_CORE = M // NC
WIN = 6400
NPASS = -(-ROWS_PER_CORE // WIN)
CLO = [min(p * WIN, ROWS_PER_CORE - WIN) for p in range(NPASS)]
SCAN_HI = CLO[1:] + [ROWS_PER_CORE]
IDX_PER_TILE = B // NS
NVREG = IDX_PER_TILE // L
KCH = 64
NCHUNK = IDX_PER_TILE // KCH + 1
NDUMMY = NS


def _body(mem_hbm, idx_hbm, val_hbm, out_hbm,
          idx_v, sel_pos, sel_off, upd, shared, sem):
    c = lax.axis_index("c")
    s = lax.axis_index("s")

    pltpu.sync_copy(idx_hbm.at[pl.ds(s * IDX_PER_TILE, IDX_PER_TILE)], idx_v)

    for p in range(NPASS):
        rows_pt = WIN // NS
        lo = c * ROWS_PER_CORE + CLO[p]
        hi = c * ROWS_PER_CORE + SCAN_HI[p]
        row0 = lo + s * rows_pt

        pltpu.sync_copy(mem_hbm.at[pl.ds(row0, rows_pt)],
                        shared.at[pl.ds(s * rows_pt, rows_pt)])
        plsc.subcore_barrier()

        def scan_body(i, cnt, lo=lo, hi=hi):
            v = idx_v[pl.ds(i * L, L)]
            m = (v >= lo) & (v < hi)
            mi = m.astype(jnp.int32)
            incl = plsc.cumsum(mi)
            lin = cnt + incl - mi
            dr = lin // KCH
            dc = lin - dr * KCH
            pos = lax.iota(jnp.int32, L) + (s * IDX_PER_TILE + i * L)
            plsc.store_scatter(sel_pos, [dr, dc], pos, mask=m)
            plsc.store_scatter(sel_off, [dr, dc], v - lo, mask=m)
            return cnt + incl[L - 1]

        cnt = lax.fori_loop(0, NVREG, scan_body, jnp.int32(0))

        dummy_pos = jnp.full((L,), s * IDX_PER_TILE, jnp.int32)
        dummy_off = jnp.full((L,), WIN + s, jnp.int32)
        for j in range(KCH // L):
            lin = cnt + j * L + lax.iota(jnp.int32, L)
            dr = lin // KCH
            dc = lin - dr * KCH
            plsc.store_scatter(sel_pos, [dr, dc], dummy_pos)
            plsc.store_scatter(sel_off, [dr, dc], dummy_off)

        def dma_body(i, carry):
            pltpu.async_copy(val_hbm.at[sel_pos.at[i]], upd, sem).wait()
            pltpu.sync_copy(upd, shared.at[sel_off.at[i]], add=True)
            return carry

        n_it = (cnt + (KCH - 1)) // KCH
        lax.fori_loop(0, n_it, dma_body, jnp.int32(0))
        plsc.subcore_barrier()

        pltpu.sync_copy(shared.at[pl.ds(s * rows_pt, rows_pt)],
                        out_hbm.at[pl.ds(row0, rows_pt)])


_sc_scatter_add = pl.kernel(
    _body,
    out_type=jax.ShapeDtypeStruct((M, D), jnp.float32),
    mesh=plsc.VectorSubcoreMesh(core_axis_name="c", subcore_axis_name="s"),
    compiler_params=pltpu.CompilerParams(needs_layout_passes=False),
    scratch_types=[
        pltpu.VMEM((IDX_PER_TILE,), jnp.int32),
        pltpu.VMEM((NCHUNK, KCH), jnp.int32),
        pltpu.VMEM((NCHUNK, KCH), jnp.int32),
        pltpu.VMEM((KCH, D), jnp.float32),
        pltpu.VMEM_SHARED((WIN + NDUMMY, D), jnp.float32),
        pltpu.SemaphoreType.DMA,
    ],
)


def kernel(mem, idx, val):
    return _sc_scatter_add(mem, idx.astype(jnp.int32), val)
